# Initial kernel scaffold; baseline (speedup 1.0000x reference)
import jax
import jax.numpy as jnp
from jax.experimental import pallas as pl


def kernel(z_loga, step):
    raise NotImplementedError("write your pallas kernel here")


# speedup vs baseline: 7.0411x; 7.0411x over previous
import functools

import jax
import jax.numpy as jnp
import numpy as np
from jax import lax
from jax.experimental import pallas as pl
from jax.experimental.pallas import tpu as pltpu
from jax.experimental.pallas import tpu_sc as plsc

ROWS = 32
COLS = 8192
K = 4096
LANES = 16
CHUNKS = COLS // LANES
NUM_CORES = 2

_SIGN = np.uint32(0x80000000)


def _hsum(v):
    iota = lax.iota(jnp.int32, LANES)
    for sh in (1, 2, 4, 8):
        v = v + jnp.take(v, (iota + sh) % LANES)
    return v[0]


def _cumsum16(v, iota):
    for sh in (1, 2, 4, 8):
        shifted = jnp.take(v, jnp.maximum(iota - sh, 0))
        v = v + jnp.where(iota >= sh, shifted, 0)
    return v


def _tec_body(z_hbm, g_hbm, out_hbm, zrow, grow, keyrow, outrow):
    wid = lax.axis_index("s") * NUM_CORES + lax.axis_index("c")

    pltpu.sync_copy(z_hbm.at[wid], zrow)
    pltpu.sync_copy(g_hbm.at[wid], grow)

    def body0(i, carry):
        sl = pl.ds(i * LANES, LANES)
        x = zrow[sl] + grow[sl]
        u = lax.bitcast_convert_type(x, jnp.uint32)
        keyrow[sl] = jnp.where(u >= _SIGN, ~u, u | _SIGN)
        return carry

    lax.fori_loop(0, CHUNKS, body0, 0)

    def count_ge(t):
        def cbody(i, acc):
            kc = keyrow[pl.ds(i * LANES, LANES)]
            return acc + jnp.where(kc >= t, 1, 0).astype(jnp.int32)

        acc = lax.fori_loop(0, CHUNKS, cbody, jnp.zeros((LANES,), jnp.int32))
        return _hsum(acc)

    def bbody(_, carry):
        prefix, bit = carry
        cand = prefix | bit
        cnt = count_ge(cand)
        prefix = jnp.where(cnt >= K, cand, prefix)
        return prefix, bit >> jnp.uint32(1)

    t, _ = lax.fori_loop(0, 32, bbody, (jnp.uint32(0), _SIGN))

    def gbody(i, acc):
        kc = keyrow[pl.ds(i * LANES, LANES)]
        return acc + jnp.where(kc > t, 1, 0).astype(jnp.int32)

    gacc = lax.fori_loop(0, CHUNKS, gbody, jnp.zeros((LANES,), jnp.int32))
    need = K - _hsum(gacc)

    iota = lax.iota(jnp.int32, LANES)

    def fbody(i, carry):
        sl = pl.ds(i * LANES, LANES)
        kc = keyrow[sl]
        gt = kc > t
        eq = kc == t
        eqi = jnp.where(eq, 1, 0).astype(jnp.int32)
        incl = _cumsum16(eqi, iota)
        sel = jnp.logical_or(gt, jnp.logical_and(eq, (carry + incl) <= need))
        outrow[sl] = jnp.where(sel, jnp.float32(1.0), jnp.float32(0.0))
        return carry + incl[LANES - 1]

    lax.fori_loop(0, CHUNKS, fbody, jnp.int32(0))

    pltpu.sync_copy(outrow, out_hbm.at[wid])


_sc_mask = functools.partial(
    pl.kernel,
    out_type=jax.ShapeDtypeStruct((ROWS, COLS), jnp.float32),
    mesh=plsc.VectorSubcoreMesh(core_axis_name="c", subcore_axis_name="s"),
    scratch_types=[
        pltpu.VMEM((COLS,), jnp.float32),
        pltpu.VMEM((COLS,), jnp.float32),
        pltpu.VMEM((COLS,), jnp.uint32),
        pltpu.VMEM((COLS,), jnp.float32),
    ],
)(_tec_body)


def kernel(z_loga, step):
    del step
    eps = jax.random.uniform(
        jax.random.key(42), z_loga.shape, z_loga.dtype, minval=1e-06, maxval=1 - 1e-06
    )
    gumbel = -jnp.log(-jnp.log(eps))
    return _sc_mask(z_loga, gumbel)


# speedup vs baseline: 11.6578x; 1.6557x over previous
import functools

import jax
import jax.numpy as jnp
import numpy as np
from jax import lax
from jax.experimental import pallas as pl
from jax.experimental.pallas import tpu as pltpu
from jax.experimental.pallas import tpu_sc as plsc

ROWS = 32
COLS = 8192
K = 4096
LANES = 16
CHUNKS = COLS // LANES
NUM_CORES = 2

_SIGN = np.uint32(0x80000000)


def _hsum(v):
    iota = lax.iota(jnp.int32, LANES)
    for sh in (1, 2, 4, 8):
        v = v + jnp.take(v, (iota + sh) % LANES)
    return v[0]


def _cumsum16(v, iota):
    for sh in (1, 2, 4, 8):
        shifted = jnp.take(v, jnp.maximum(iota - sh, 0))
        v = v + jnp.where(iota >= sh, shifted, 0)
    return v


def _tec_body(z_hbm, g_hbm, out_hbm, zrow, grow, keyrow, outrow):
    wid = lax.axis_index("s") * NUM_CORES + lax.axis_index("c")

    pltpu.sync_copy(z_hbm.at[wid], zrow)
    pltpu.sync_copy(g_hbm.at[wid], grow)

    def body0(i, carry):
        sl = pl.ds(i * LANES, LANES)
        x = zrow[sl] + grow[sl]
        u = lax.bitcast_convert_type(x, jnp.uint32)
        keyrow[sl] = jnp.where(u >= _SIGN, ~u, u | _SIGN)
        return carry

    lax.fori_loop(0, CHUNKS, body0, 0)

    def radix_pass(prefix, upper, shift, nbits):
        ncand = (1 << nbits) - 1

        def cbody(i, accs):
            kc = keyrow[pl.ds(i * LANES, LANES)]
            return tuple(
                accs[j - 1]
                + jnp.where(kc >= (prefix | (np.uint32(j) << shift)), 1, 0).astype(
                    jnp.int32
                )
                for j in range(1, ncand + 1)
            )

        accs = lax.fori_loop(
            0,
            CHUNKS,
            cbody,
            tuple(jnp.zeros((LANES,), jnp.int32) for _ in range(ncand)),
        )
        cnts = [_hsum(a) for a in accs]
        newprefix = prefix
        for j in range(1, ncand + 1):
            newprefix = jnp.where(
                cnts[j - 1] >= K, prefix | (np.uint32(j) << shift), newprefix
            )
        below = [jnp.where(c < K, c, -1) for c in cnts]
        mx = below[0]
        for b in below[1:]:
            mx = jnp.maximum(mx, b)
        upper = jnp.where(mx >= 0, mx, upper)
        return newprefix, upper

    prefix = jnp.uint32(0)
    upper = jnp.int32(0)
    for shift in list(range(29, -1, -3)) + [0]:
        prefix, upper = radix_pass(prefix, upper, np.uint32(shift), 3 if shift else 2)
    t = prefix
    need = K - upper

    iota = lax.iota(jnp.int32, LANES)

    def fbody(i, carry):
        sl = pl.ds(i * LANES, LANES)
        kc = keyrow[sl]
        gt = kc > t
        eq = kc == t
        eqi = jnp.where(eq, 1, 0).astype(jnp.int32)
        incl = _cumsum16(eqi, iota)
        sel = jnp.logical_or(gt, jnp.logical_and(eq, (carry + incl) <= need))
        outrow[sl] = jnp.where(sel, jnp.float32(1.0), jnp.float32(0.0))
        return carry + incl[LANES - 1]

    lax.fori_loop(0, CHUNKS, fbody, jnp.int32(0))

    pltpu.sync_copy(outrow, out_hbm.at[wid])


_sc_mask = functools.partial(
    pl.kernel,
    out_type=jax.ShapeDtypeStruct((ROWS, COLS), jnp.float32),
    mesh=plsc.VectorSubcoreMesh(core_axis_name="c", subcore_axis_name="s"),
    scratch_types=[
        pltpu.VMEM((COLS,), jnp.float32),
        pltpu.VMEM((COLS,), jnp.float32),
        pltpu.VMEM((COLS,), jnp.uint32),
        pltpu.VMEM((COLS,), jnp.float32),
    ],
)(_tec_body)


def kernel(z_loga, step):
    del step
    eps = jax.random.uniform(
        jax.random.key(42), z_loga.shape, z_loga.dtype, minval=1e-06, maxval=1 - 1e-06
    )
    gumbel = -jnp.log(-jnp.log(eps))
    return _sc_mask(z_loga, gumbel)
